# baseline (speedup 1.0000x reference)
import jax
import jax.numpy as jnp
from jax.experimental import pallas as pl
from jax.experimental.pallas import tpu as pltpu

_SPLIT = 4
_ALIGN = 8


def _body(patches_sm, s1_hbm, s2_hbm, geo_ref, out_ref,
          rows_buf, cols_buf, rsem, csem):
    hp = geo_ref.shape[1]
    ps = geo_ref.shape[2]
    d = s1_hbm.shape[2]
    win = ps + _ALIGN
    nsteps = _SPLIT * s1_hbm.shape[0]
    i = pl.program_id(0)

    def window_copies(step, slot):
        b = step // _SPLIT
        h = step % _SPLIT
        copies = []
        for p in range(hp):
            r0 = patches_sm[b, h * hp + p, 0]
            c0 = patches_sm[b, h * hp + p, 1]
            r_al = pl.multiple_of((r0 // _ALIGN) * _ALIGN, _ALIGN)
            c_al = pl.multiple_of((c0 // _ALIGN) * _ALIGN, _ALIGN)
            copies.append(pltpu.make_async_copy(
                s1_hbm.at[b, pl.ds(r_al, win), :],
                rows_buf.at[slot, p], rsem.at[slot]))
            copies.append(pltpu.make_async_copy(
                s2_hbm.at[b, pl.ds(c_al, win), :],
                cols_buf.at[slot, p], csem.at[slot]))
        return copies

    slot = i % 2
    nxt = (i + 1) % 2

    @pl.when(i == 0)
    def _():
        for cp in window_copies(i, slot):
            cp.start()

    @pl.when(i + 1 < nsteps)
    def _():
        for cp in window_copies(i + 1, nxt):
            cp.start()

    for cp in window_copies(i, slot):
        cp.wait()

    b = i // _SPLIT
    h = i % _SPLIT
    for p in range(hp):
        rr = patches_sm[b, h * hp + p, 0] % _ALIGN
        cc = patches_sm[b, h * hp + p, 1] % _ALIGN
        rows = rows_buf[slot, p, pl.ds(rr, ps), :]
        cols = cols_buf[slot, p, pl.ds(cc, ps), :]
        rc = jnp.concatenate(
            [jnp.broadcast_to(rows[None, :, :], (ps, ps, d)),
             jnp.broadcast_to(cols[:, None, :], (ps, ps, d))], axis=-1)
        out_ref[0, p, :, :, 0:2 * d] = rc
        out_ref[0, p, :, :, 2 * d:2 * d + 1] = geo_ref[0, p][..., None]


def kernel(seq1M, seq2M, patches, geo):
    B, SR, D = seq1M.shape
    SL = seq2M.shape[1]
    P = patches.shape[1]
    PS = geo.shape[2]
    C = 2 * D + 1
    HP = P // _SPLIT
    WIN = PS + _ALIGN

    grid_spec = pltpu.PrefetchScalarGridSpec(
        num_scalar_prefetch=1,
        grid=(_SPLIT * B,),
        in_specs=[
            pl.BlockSpec(memory_space=pltpu.MemorySpace.HBM),
            pl.BlockSpec(memory_space=pltpu.MemorySpace.HBM),
            pl.BlockSpec((1, HP, PS, PS),
                         lambda i, pref: (i // _SPLIT, i % _SPLIT, 0, 0)),
        ],
        out_specs=pl.BlockSpec((1, HP, PS, PS, C),
                               lambda i, pref: (i // _SPLIT, i % _SPLIT,
                                                0, 0, 0)),
        scratch_shapes=[
            pltpu.VMEM((2, HP, WIN, D), jnp.float32),
            pltpu.VMEM((2, HP, WIN, D), jnp.float32),
            pltpu.SemaphoreType.DMA((2,)),
            pltpu.SemaphoreType.DMA((2,)),
        ],
    )
    return pl.pallas_call(
        _body,
        grid_spec=grid_spec,
        out_shape=jax.ShapeDtypeStruct((B, P, PS, PS, C), jnp.float32),
        compiler_params=pltpu.CompilerParams(
            dimension_semantics=("arbitrary",),
            vmem_limit_bytes=60 * 1024 * 1024),
    )(patches, seq1M, seq2M, geo)


# speedup vs baseline: 1.1463x; 1.1463x over previous
import jax
import jax.numpy as jnp
from jax.experimental import pallas as pl
from jax.experimental.pallas import tpu as pltpu

_SPLIT = 2
_ALIGN = 8


def _body(patches_sm, s1_hbm, s2_hbm, geo_ref, out_ref,
          rows_buf, cols_buf, rsem, csem):
    hp = geo_ref.shape[1]
    ps = geo_ref.shape[2]
    d = s1_hbm.shape[2]
    win = ps + _ALIGN
    nsteps = _SPLIT * s1_hbm.shape[0]
    i = pl.program_id(0)

    def window_copies(step, slot):
        b = step // _SPLIT
        h = step % _SPLIT
        copies = []
        for p in range(hp):
            r0 = patches_sm[b, h * hp + p, 0]
            c0 = patches_sm[b, h * hp + p, 1]
            r_al = pl.multiple_of((r0 // _ALIGN) * _ALIGN, _ALIGN)
            c_al = pl.multiple_of((c0 // _ALIGN) * _ALIGN, _ALIGN)
            copies.append(pltpu.make_async_copy(
                s1_hbm.at[b, pl.ds(r_al, win), :],
                rows_buf.at[slot, p], rsem.at[slot]))
            copies.append(pltpu.make_async_copy(
                s2_hbm.at[b, pl.ds(c_al, win), :],
                cols_buf.at[slot, p], csem.at[slot]))
        return copies

    slot = i % 2
    nxt = (i + 1) % 2

    @pl.when(i == 0)
    def _():
        for cp in window_copies(i, slot):
            cp.start()

    @pl.when(i + 1 < nsteps)
    def _():
        for cp in window_copies(i + 1, nxt):
            cp.start()

    for cp in window_copies(i, slot):
        cp.wait()

    b = i // _SPLIT
    h = i % _SPLIT
    for p in range(hp):
        rr = patches_sm[b, h * hp + p, 0] % _ALIGN
        cc = patches_sm[b, h * hp + p, 1] % _ALIGN
        rows = rows_buf[slot, p, pl.ds(rr, ps), :]
        cols = cols_buf[slot, p, pl.ds(cc, ps), :]
        rc = jnp.concatenate(
            [jnp.broadcast_to(rows[None, :, :], (ps, ps, d)),
             jnp.broadcast_to(cols[:, None, :], (ps, ps, d))], axis=-1)
        out_ref[0, p, :, :, 0:2 * d] = rc
        out_ref[0, p, :, :, 2 * d:2 * d + 1] = geo_ref[0, p][..., None]


def kernel(seq1M, seq2M, patches, geo):
    B, SR, D = seq1M.shape
    SL = seq2M.shape[1]
    P = patches.shape[1]
    PS = geo.shape[2]
    C = 2 * D + 1
    HP = P // _SPLIT
    WIN = PS + _ALIGN

    grid_spec = pltpu.PrefetchScalarGridSpec(
        num_scalar_prefetch=1,
        grid=(_SPLIT * B,),
        in_specs=[
            pl.BlockSpec(memory_space=pltpu.MemorySpace.HBM),
            pl.BlockSpec(memory_space=pltpu.MemorySpace.HBM),
            pl.BlockSpec((1, HP, PS, PS),
                         lambda i, pref: (i // _SPLIT, i % _SPLIT, 0, 0)),
        ],
        out_specs=pl.BlockSpec((1, HP, PS, PS, C),
                               lambda i, pref: (i // _SPLIT, i % _SPLIT,
                                                0, 0, 0)),
        scratch_shapes=[
            pltpu.VMEM((2, HP, WIN, D), jnp.float32),
            pltpu.VMEM((2, HP, WIN, D), jnp.float32),
            pltpu.SemaphoreType.DMA((2,)),
            pltpu.SemaphoreType.DMA((2,)),
        ],
    )
    return pl.pallas_call(
        _body,
        grid_spec=grid_spec,
        out_shape=jax.ShapeDtypeStruct((B, P, PS, PS, C), jnp.float32),
        compiler_params=pltpu.CompilerParams(
            dimension_semantics=("arbitrary",),
            vmem_limit_bytes=60 * 1024 * 1024),
    )(patches, seq1M, seq2M, geo)
